# baseline (speedup 1.0000x reference)
import functools

import jax
import jax.numpy as jnp
from jax import lax
from jax.experimental import pallas as pl
from jax.experimental.pallas import tpu as pltpu
from jax.experimental.pallas import tpu_sc as plsc

NUM_FIELDS = 26
FIELD_DIM = 38461
D = 16
B = 4096
NNZ = 26
NC = 2
NS = 16
NW = NC * NS
ROWS_PT = B // NW
EW = 32
E_PT = ROWS_PT * EW
CH = 128
NCH = E_PT // CH
GRP = 8
NG = NCH // GRP
OUT_PT = ROWS_PT * (NUM_FIELDS - 1)
SC_ROWS = NS * OUT_PT
XCOL = 0
FCOL = 32


@functools.partial(
    pl.kernel,
    out_type=jax.ShapeDtypeStruct((B * (NUM_FIELDS - 1), D), jnp.float32),
    mesh=plsc.VectorSubcoreMesh(core_axis_name="c", subcore_axis_name="s"),
    compiler_params=pltpu.CompilerParams(use_tc_tiling_on_sc=False,
                                         needs_layout_passes=False),
    scratch_types=[
        pltpu.VMEM((ROWS_PT, 128), jnp.int32),
        pltpu.VMEM((NCH, CH), jnp.int32),
        pltpu.VMEM((NCH, CH), jnp.int32),
        pltpu.VMEM((2, GRP * CH, D), jnp.float32),
        pltpu.VMEM_SHARED((SC_ROWS, D), jnp.float32),
        pltpu.SemaphoreType.DMA,
        pltpu.SemaphoreType.DMA,
        pltpu.SemaphoreType.DMA,
        pltpu.SemaphoreType.DMA,
        pltpu.SemaphoreType.DMA,
    ],
)
def _emb(xc_hbm, table_hbm, zeros_hbm, out_hbm, xc_v, gidx_v, d_v, rows_v,
         acc_sh, semg, *sema):
    sid = lax.axis_index("s")
    wid = sid * NC + lax.axis_index("c")
    pltpu.sync_copy(xc_hbm.at[pl.ds(wid * ROWS_PT, ROWS_PT)], xc_v)

    obase = sid * OUT_PT
    for r in range(ROWS_PT):
        for h in range(2):
            xv = xc_v[r, pl.ds(XCOL + h * 16, 16)]
            f = xc_v[r, pl.ds(FCOL + h * 16, 16)]
            nz = jnp.minimum(f, 1)
            gid = (xv + f * FIELD_DIM) * nz
            d = (obase + r * (NUM_FIELDS - 1)) + f - nz
            e = r * EW + h * 16
            gidx_v[e // CH, pl.ds(e % CH, 16)] = gid
            d_v[e // CH, pl.ds(e % CH, 16)] = d

    adds = [[] for _ in range(NG)]
    for g in range(NG):
        p = g % 2
        for a in adds[g - 2] if g >= 2 else ():
            a.wait()
        gathers = [
            pltpu.async_copy(
                table_hbm.at[gidx_v.at[g * GRP + k]],
                rows_v.at[p, pl.ds(k * CH, CH)], semg)
            for k in range(GRP)
        ]
        if g == 0:
            pltpu.sync_copy(zeros_hbm, acc_sh.at[pl.ds(obase, OUT_PT)])
        for gth in gathers:
            gth.wait()
        adds[g] = [
            pltpu.async_copy(
                rows_v.at[p, pl.ds(k * CH, CH)],
                acc_sh.at[d_v.at[g * GRP + k]], sema[g], add=True)
            for k in range(GRP)
        ]
    for g in (NG - 2, NG - 1):
        for a in adds[g]:
            a.wait()

    pltpu.sync_copy(acc_sh.at[pl.ds(obase, OUT_PT)],
                    out_hbm.at[pl.ds(wid * OUT_PT, OUT_PT)])


def kernel(x_field, x, table):
    xf = x_field.astype(jnp.int32)
    xx = x.astype(jnp.int32)
    zc = jnp.zeros((B, 6), jnp.int32)
    zt = jnp.zeros((B, 128 - 2 * EW), jnp.int32)
    xcomb = jnp.concatenate([xx, zc, xf, zc, zt], axis=1)
    zeros = jnp.zeros((OUT_PT, D), jnp.float32)
    out = _emb(xcomb, table, zeros)
    return out.reshape(B, NUM_FIELDS - 1, D)


# speedup vs baseline: 1.0989x; 1.0989x over previous
import functools

import jax
import jax.numpy as jnp
from jax import lax
from jax.experimental import pallas as pl
from jax.experimental.pallas import tpu as pltpu
from jax.experimental.pallas import tpu_sc as plsc

NUM_FIELDS = 26
FIELD_DIM = 38461
D = 16
B = 4096
NNZ = 26
NC = 2
NS = 16
NW = NC * NS
ROWS_PT = B // NW
EW = 32
E_PT = ROWS_PT * EW
CH = 128
NCH = E_PT // CH
GRP = 16
NG = NCH // GRP
ACC_R = (NUM_FIELDS - 1) * D
XCOL = 0
FCOL = 32


@functools.partial(
    pl.kernel,
    out_type=jax.ShapeDtypeStruct((ACC_R, B), jnp.float32),
    mesh=plsc.VectorSubcoreMesh(core_axis_name="c", subcore_axis_name="s"),
    compiler_params=pltpu.CompilerParams(use_tc_tiling_on_sc=False,
                                         needs_layout_passes=False),
    scratch_types=[
        pltpu.VMEM((ROWS_PT * 128,), jnp.int32),
        pltpu.VMEM((NCH, CH), jnp.int32),
        pltpu.VMEM((E_PT,), jnp.int32),
        pltpu.VMEM((GRP * CH, D), jnp.float32),
        pltpu.VMEM((ACC_R, ROWS_PT), jnp.float32),
        pltpu.SemaphoreType.DMA,
    ],
)
def _emb(xc_hbm, table_hbm, zeros_hbm, out_hbm, xc_v, gidx_v, d_v, rows_v,
         acc_v, semg):
    wid = lax.axis_index("s") * NC + lax.axis_index("c")
    pltpu.sync_copy(xc_hbm.at[pl.ds(wid * ROWS_PT * 128, ROWS_PT * 128)],
                    xc_v)
    zinit = pltpu.async_copy(zeros_hbm, acc_v, semg)

    iota = lax.iota(jnp.int32, 16)
    for r in range(ROWS_PT):
        for h in range(2):
            xv = xc_v[pl.ds(r * 128 + XCOL + h * 16, 16)]
            f = xc_v[pl.ds(r * 128 + FCOL + h * 16, 16)]
            nz = jnp.minimum(f, 1)
            gid = (xv + f * FIELD_DIM) * nz
            d = (f - nz) * D
            e = r * EW + h * 16
            gidx_v[e // CH, pl.ds(e % CH, 16)] = gid
            d_v[pl.ds(e, 16)] = d

    zinit.wait()

    for g in range(NG):
        gathers = [
            pltpu.async_copy(
                table_hbm.at[gidx_v.at[g * GRP + k]],
                rows_v.at[pl.ds(k * CH, CH)], semg)
            for k in range(GRP)
        ]
        for c in gathers:
            c.wait()
        base = g * GRP * CH // 16

        def abody(i, carry, _base=base):
            dvec = d_v[pl.ds((_base + i) * 16, 16)]
            colv = jnp.full((16,), (_base + i) // 2, jnp.int32)
            for lane in range(16):
                e = i * 16 + lane
                vals = plsc.load_gather(
                    rows_v, [jnp.full((16,), e, jnp.int32), iota])
                plsc.addupdate_scatter(acc_v, [dvec[lane] + iota, colv],
                                       vals)
            return carry

        lax.fori_loop(0, GRP * CH // 16, abody, 0)

    pltpu.sync_copy(acc_v, out_hbm.at[:, pl.ds(wid * ROWS_PT, ROWS_PT)])


def kernel(x_field, x, table):
    xf = x_field.astype(jnp.int32)
    xx = x.astype(jnp.int32)
    zc = jnp.zeros((B, 6), jnp.int32)
    zt = jnp.zeros((B, 128 - 2 * EW), jnp.int32)
    xcomb = jnp.concatenate([xx, zc, xf, zc, zt], axis=1).reshape(-1)
    zeros = jnp.zeros((ACC_R, ROWS_PT), jnp.float32)
    out_t = _emb(xcomb, table, zeros)
    return out_t.reshape(NUM_FIELDS - 1, D, B).transpose(2, 0, 1)
